# baseline (speedup 1.0000x reference)
import functools

import jax
import jax.numpy as jnp
import numpy as np
from jax import lax
from jax.experimental import pallas as pl
from jax.experimental.pallas import tpu as pltpu
from jax.experimental.pallas import tpu_sc as plsc

TOP_K = 30
NUM_RBF = 16
NUM_PE = 16
B, N = 8, 1024
NE = B * N * TOP_K
_INF = np.float32(np.inf)


def _d2_body(xr_ref, xc_ref, o_ref):
    xr = xr_ref[0]
    xc = xc_ref[0]
    acc = None
    for c in range(3):
        d = xr[:, c:c + 1] - xc[c:c + 1, :]
        acc = d * d if acc is None else acc + d * d
    o_ref[0] = acc


def _pairwise_d2(x, xT):
    return pl.pallas_call(
        _d2_body,
        grid=(B, 4),
        in_specs=[
            pl.BlockSpec((1, 256, 3), lambda b, j: (b, j, 0)),
            pl.BlockSpec((1, 3, N), lambda b, j: (b, 0, 0)),
        ],
        out_specs=pl.BlockSpec((1, 256, N), lambda b, j: (b, j, 0)),
        out_shape=jax.ShapeDtypeStruct((B, N, N), jnp.float32),
    )(x, xT)


def _rev(v):
    return lax.rev(v, (0,))


def _merge16_keys(a, b):
    rb = _rev(b)
    lo = jnp.minimum(a, rb)
    hi = jnp.maximum(a, rb)
    return lax.sort(lo), lax.sort(hi)


def _mergelow32_keys(a, b):
    a0, a1 = a
    b0, b1 = b
    c0 = jnp.minimum(a0, _rev(b1))
    c1 = jnp.minimum(a1, _rev(b0))
    d0 = jnp.minimum(c0, c1)
    d1 = jnp.maximum(c0, c1)
    return lax.sort(d0), lax.sort(d1)


def _merge16_kv(ak, av, bk, bv):
    rk, rv = _rev(bk), _rev(bv)
    c = ak <= rk
    lo_k = jnp.where(c, ak, rk)
    lo_v = jnp.where(c, av, rv)
    hi_k = jnp.where(c, rk, ak)
    hi_v = jnp.where(c, rv, av)
    s0 = plsc.sort_key_val(lo_k, lo_v)
    s1 = plsc.sort_key_val(hi_k, hi_v)
    return s0[0], s0[1], s1[0], s1[1]


def _topk_sc(d2):
    mesh = plsc.VectorSubcoreMesh(core_axis_name="c", subcore_axis_name="s",
                                  num_cores=2, num_subcores=16)

    @functools.partial(
        pl.kernel,
        out_type=[
            jax.ShapeDtypeStruct((NE,), jnp.int32),
            jax.ShapeDtypeStruct((NE,), jnp.float32),
        ],
        mesh=mesh,
        compiler_params=pltpu.CompilerParams(needs_layout_passes=False),
        scratch_types=[
            pltpu.VMEM((32 * N,), jnp.float32),
            pltpu.VMEM((1024,), jnp.float32),
            pltpu.VMEM((1024,), jnp.int32),
            pltpu.VMEM((976,), jnp.int32),
            pltpu.VMEM((976,), jnp.float32),
        ],
    )
    def k(d2_hbm, eidx_hbm, d2sel_hbm, d2buf, cand_k, cand_v, eidx_st, d2st):
        iota16 = lax.iota(jnp.int32, 16)
        w = lax.axis_index("s") * 2 + lax.axis_index("c")

        def subchunk(s, _):
            base_row = w * 256 + s * 32
            pltpu.sync_copy(d2_hbm.at[pl.ds(base_row * N, 32 * N)], d2buf)

            def rowbody(r, __):
                rb = r * N
                m = [None] * 4
                for c in range(64):
                    v = d2buf[pl.ds(rb + c * 16, 16)]
                    q = c // 16
                    m[q] = v if m[q] is None else jnp.minimum(m[q], v)
                sm = [lax.sort(mi) for mi in m]
                ab = _merge16_keys(sm[0], sm[1])
                cd = _merge16_keys(sm[2], sm[3])
                low = _mergelow32_keys(ab, cd)
                t = low[1][13]
                off = jnp.int32(0)
                for c in range(64):
                    v = d2buf[pl.ds(rb + c * 16, 16)]
                    msk = v <= t
                    plsc.store_compressed(cand_k.at[pl.ds(off, 16)], v,
                                          mask=msk)
                    plsc.store_compressed(cand_v.at[pl.ds(off, 16)],
                                          iota16 + (c * 16), mask=msk)
                    cnt = plsc.all_reduce_population_count(msk)
                    off = off + cnt[0]
                k0 = cand_k[pl.ds(0, 16)]
                v0 = cand_v[pl.ds(0, 16)]
                k0, v0 = plsc.sort_key_val(k0, v0)
                k1 = cand_k[pl.ds(16, 16)]
                v1 = cand_v[pl.ds(16, 16)]
                k1 = jnp.where(iota16 < (off - 16), k1, _INF)
                k1, v1 = plsc.sort_key_val(k1, v1)
                state = _merge16_kv(k0, v0, k1, v1)

                def cbody(i, st):
                    s0k, s0v, s1k, s1v = st
                    ck = cand_k[pl.ds(i * 16, 16)]
                    cv = cand_v[pl.ds(i * 16, 16)]
                    ck = jnp.where(iota16 < (off - i * 16), ck, _INF)
                    ck, cv = plsc.sort_key_val(ck, cv)
                    rk, rv = _rev(ck), _rev(cv)
                    cc = s1k <= rk
                    ek = jnp.where(cc, s1k, rk)
                    ev = jnp.where(cc, s1v, rv)
                    ek, ev = plsc.sort_key_val(ek, ev)
                    return _merge16_kv(s0k, s0v, ek, ev)

                nv = (off + 15) // 16
                s0k, s0v, s1k, s1v = lax.fori_loop(2, nv, cbody, state)

                pb = r * 30
                eidx_st[pl.ds(pb, 16)] = s0v
                d2st[pl.ds(pb, 16)] = s0k
                m14 = iota16 < 14
                plsc.store_compressed(eidx_st.at[pl.ds(pb + 16, 16)], s1v,
                                      mask=m14)
                plsc.store_compressed(d2st.at[pl.ds(pb + 16, 16)], s1k,
                                      mask=m14)
                return __

            lax.fori_loop(0, 32, rowbody, 0)
            base = w * 7680 + s * 960
            pltpu.sync_copy(eidx_st.at[pl.ds(0, 960)],
                            eidx_hbm.at[pl.ds(base, 960)])
            pltpu.sync_copy(d2st.at[pl.ds(0, 960)],
                            d2sel_hbm.at[pl.ds(base, 960)])
            return _

        lax.fori_loop(0, 8, subchunk, 0)

    return k(d2.reshape(-1))


def _l2norm(v, axis=-1, eps=1e-12):
    n = jnp.sqrt(jnp.sum(v * v, axis=axis, keepdims=True))
    return v / jnp.maximum(n, eps)


def _gather_nodes(nodes, neighbor_idx):
    Bb, Nn, K = neighbor_idx.shape
    idx = neighbor_idx.reshape(Bb, Nn * K)[:, :, None]
    out = jnp.take_along_axis(nodes, idx, axis=1)
    return out.reshape(Bb, Nn, K, nodes.shape[-1])


def _rbf(D):
    D_mu = jnp.linspace(0.0, 20.0, NUM_RBF).reshape(1, 1, 1, -1)
    D_sigma = 20.0 / NUM_RBF
    return jnp.exp(-(((D[..., None] - D_mu) / D_sigma) ** 2))


def _quaternions(R):
    diag = jnp.diagonal(R, axis1=-2, axis2=-1)
    Rxx, Ryy, Rzz = diag[..., 0], diag[..., 1], diag[..., 2]
    magnitudes = 0.5 * jnp.sqrt(jnp.abs(1 + jnp.stack([Rxx - Ryy - Rzz, -Rxx + Ryy - Rzz, -Rxx - Ryy + Rzz], axis=-1)))
    signs = jnp.sign(jnp.stack([R[..., 2, 1] - R[..., 1, 2], R[..., 0, 2] - R[..., 2, 0], R[..., 1, 0] - R[..., 0, 1]], axis=-1))
    xyz = signs * magnitudes
    w = jnp.sqrt(jax.nn.relu(1 + jnp.sum(diag, axis=-1, keepdims=True))) / 2.0
    Q = jnp.concatenate([xyz, w], axis=-1)
    return _l2norm(Q)


def _orientations(x, edge_idx):
    dX = x[:, 1:, :] - x[:, :-1, :]
    U = _l2norm(dX)
    u_2 = U[:, :-2, :]
    u_1 = U[:, 1:-1, :]
    n_2 = _l2norm(jnp.cross(u_2, u_1))
    o_1 = _l2norm(u_2 - u_1)
    O = jnp.stack([o_1, n_2, jnp.cross(o_1, n_2)], axis=2)
    O = O.reshape(O.shape[0], O.shape[1], 9)
    O = jnp.pad(O, ((0, 0), (1, 2), (0, 0)))
    O_neighbors = _gather_nodes(O, edge_idx)
    X_neighbors = _gather_nodes(x, edge_idx)
    Bb, Nn = O.shape[0], O.shape[1]
    K = edge_idx.shape[2]
    O = O.reshape(Bb, Nn, 3, 3)
    O_neighbors = O_neighbors.reshape(Bb, Nn, K, 3, 3)
    dXn = X_neighbors - x[:, :, None, :]
    dU = jnp.matmul(O[:, :, None], dXn[..., None])[..., 0]
    dU = _l2norm(dU)
    R = jnp.matmul(jnp.swapaxes(O[:, :, None], -1, -2), O_neighbors)
    Q = _quaternions(R)
    return jnp.concatenate([dU, Q], axis=-1)


def _pe(edge_idx):
    n_nodes = edge_idx.shape[1]
    ii = jnp.arange(n_nodes, dtype=jnp.float32).reshape(1, -1, 1)
    d = (edge_idx.astype(jnp.float32) - ii)[..., None]
    frequency = jnp.exp(jnp.arange(0, NUM_PE, 2, dtype=jnp.float32) * (-(np.log(10000.0) / NUM_PE)))
    angles = d * frequency.reshape(1, 1, 1, -1)
    return jnp.concatenate([jnp.cos(angles), jnp.sin(angles)], axis=-1)


def _linear_ln_body(e_ref, wt_ref, b_ref, g_ref, beta_ref, o_ref):
    e = e_ref[...]
    y = jnp.dot(e, wt_ref[...], preferred_element_type=jnp.float32) + b_ref[...]
    mu = jnp.mean(y, axis=-1, keepdims=True)
    d = y - mu
    var = jnp.sum(d * d, axis=-1, keepdims=True) * (1.0 / (y.shape[-1] - 1))
    sigma = jnp.sqrt(var + 1e-6)
    o_ref[...] = g_ref[...] * d / (sigma + 1e-6) + beta_ref[...]


def _linear_ln(E_feat, W, b, gain, bias):
    M, F = E_feat.shape
    OUT = W.shape[0]
    BM = 2048
    return pl.pallas_call(
        _linear_ln_body,
        grid=(M // BM,),
        in_specs=[
            pl.BlockSpec((BM, F), lambda i: (i, 0)),
            pl.BlockSpec((F, OUT), lambda i: (0, 0)),
            pl.BlockSpec((1, OUT), lambda i: (0, 0)),
            pl.BlockSpec((1, OUT), lambda i: (0, 0)),
            pl.BlockSpec((1, OUT), lambda i: (0, 0)),
        ],
        out_specs=pl.BlockSpec((BM, OUT), lambda i: (i, 0)),
        out_shape=jax.ShapeDtypeStruct((M, OUT), jnp.float32),
    )(E_feat, W.T, b.reshape(1, OUT), gain.reshape(1, OUT), bias.reshape(1, OUT))


def kernel(x, mask, W, b, gain, bias):
    xT = jnp.transpose(x, (0, 2, 1))
    d2 = _pairwise_d2(x, xT)
    eidx_flat, d2sel_flat = _topk_sc(d2)
    edge_idx = eidx_flat.reshape(B, N, TOP_K)
    D_neighbors = jnp.sqrt(d2sel_flat.reshape(B, N, TOP_K) + 1e-6)
    rbf = _rbf(D_neighbors)
    o_features = _orientations(x, edge_idx)
    e_positional = _pe(edge_idx)
    E = jnp.concatenate([e_positional, rbf, o_features], axis=-1)
    F = E.shape[-1]
    out = _linear_ln(E.reshape(NE, F), W, b, gain, bias)
    return out.reshape(B, N, TOP_K, W.shape[0]), edge_idx


# speedup vs baseline: 11.3503x; 11.3503x over previous
import functools

import jax
import jax.numpy as jnp
import numpy as np
from jax import lax
from jax.experimental import pallas as pl
from jax.experimental.pallas import tpu as pltpu
from jax.experimental.pallas import tpu_sc as plsc

TOP_K = 30
B, N = 8, 1024
NE = B * N * TOP_K
NPL = 26
_INF = np.float32(np.inf)

_FREQS = (np.arange(0, 16, 2, dtype=np.float32)
          * np.float32(-(np.log(10000.0) / 16.0)))
_FREQS = np.exp(_FREQS).astype(np.float32)
_MUS = np.linspace(0.0, 20.0, 16).astype(np.float32)
_SIG = np.float32(1.25)
_PI2_HI = np.float32(2.0 * np.pi)
_PI2_LO = np.float32(2.0 * np.pi - np.float64(_PI2_HI))
_INV_2PI = np.float32(1.0 / (2.0 * np.pi))


def _safe_inv_norm(n2):
    n = jnp.sqrt(n2)
    return 1.0 / jnp.maximum(n, 1e-12)


def _d2_body(xr_ref, xc_ref, o_ref, t_ref):
    j = pl.program_id(1)
    xr = xr_ref[0]
    xc = xc_ref[0]
    acc = None
    for c in range(3):
        d = xr[:, c:c + 1] - xc[c:c + 1, :]
        acc = d * d if acc is None else acc + d * d
    o_ref[0] = acc

    @pl.when(j == 0)
    def _():
        t_ref[0] = jnp.zeros((16, N), jnp.float32)
        xx = [xc[c:c + 1, :] for c in range(3)]
        dx = [xx[c][:, 1:N] - xx[c][:, 0:N - 1] for c in range(3)]
        inv = _safe_inv_norm(dx[0] * dx[0] + dx[1] * dx[1] + dx[2] * dx[2])
        u = [dx[c] * inv for c in range(3)]
        u2 = [u[c][:, 0:N - 3] for c in range(3)]
        u1 = [u[c][:, 1:N - 2] for c in range(3)]
        cr = [u2[1] * u1[2] - u2[2] * u1[1],
              u2[2] * u1[0] - u2[0] * u1[2],
              u2[0] * u1[1] - u2[1] * u1[0]]
        inv = _safe_inv_norm(cr[0] * cr[0] + cr[1] * cr[1] + cr[2] * cr[2])
        n2v = [cr[c] * inv for c in range(3)]
        o1r = [u2[c] - u1[c] for c in range(3)]
        inv = _safe_inv_norm(o1r[0] * o1r[0] + o1r[1] * o1r[1]
                             + o1r[2] * o1r[2])
        o1 = [o1r[c] * inv for c in range(3)]
        th = [o1[1] * n2v[2] - o1[2] * n2v[1],
              o1[2] * n2v[0] - o1[0] * n2v[2],
              o1[0] * n2v[1] - o1[1] * n2v[0]]
        rows = o1 + n2v + th
        for ri, val in enumerate(rows):
            t_ref[0, ri:ri + 1, 1:N - 2] = val
        for c in range(3):
            t_ref[0, 9 + c:10 + c, :] = xx[c]


def _pairwise_d2(x, xT):
    return pl.pallas_call(
        _d2_body,
        grid=(B, 4),
        in_specs=[
            pl.BlockSpec((1, 256, 3), lambda b, j: (b, j, 0)),
            pl.BlockSpec((1, 3, N), lambda b, j: (b, 0, 0)),
        ],
        out_specs=[
            pl.BlockSpec((1, 256, N), lambda b, j: (b, j, 0)),
            pl.BlockSpec((1, 16, N), lambda b, j: (b, 0, 0)),
        ],
        out_shape=[
            jax.ShapeDtypeStruct((B, N, N), jnp.float32),
            jax.ShapeDtypeStruct((B, 16, N), jnp.float32),
        ],
    )(x, xT)


def _rev(v):
    return lax.rev(v, (0,))


def _merge16_keys(a, b):
    rb = _rev(b)
    lo = jnp.minimum(a, rb)
    hi = jnp.maximum(a, rb)
    return lax.sort(lo), lax.sort(hi)


def _mergelow32_keys(a, b):
    a0, a1 = a
    b0, b1 = b
    c0 = jnp.minimum(a0, _rev(b1))
    c1 = jnp.minimum(a1, _rev(b0))
    d0 = jnp.minimum(c0, c1)
    d1 = jnp.maximum(c0, c1)
    return lax.sort(d0), lax.sort(d1)


def _merge16_kv(ak, av, bk, bv):
    rk, rv = _rev(bk), _rev(bv)
    c = ak <= rk
    lo_k = jnp.where(c, ak, rk)
    lo_v = jnp.where(c, av, rv)
    hi_k = jnp.where(c, rk, ak)
    hi_v = jnp.where(c, rv, av)
    s0 = plsc.sort_key_val(lo_k, lo_v)
    s1 = plsc.sort_key_val(hi_k, hi_v)
    return s0[0], s0[1], s1[0], s1[1]


def _topk_sc(d2_flat, t_flat):
    mesh = plsc.VectorSubcoreMesh(core_axis_name="c", subcore_axis_name="s",
                                  num_cores=2, num_subcores=16)

    @functools.partial(
        pl.kernel,
        out_type=[
            jax.ShapeDtypeStruct((NE,), jnp.int32),
            jax.ShapeDtypeStruct((NPL * NE,), jnp.float32),
        ],
        mesh=mesh,
        compiler_params=pltpu.CompilerParams(needs_layout_passes=False),
        scratch_types=[
            pltpu.VMEM((32 * N,), jnp.float32),
            pltpu.VMEM((16 * N,), jnp.float32),
            pltpu.VMEM((1024,), jnp.float32),
            pltpu.VMEM((1024,), jnp.int32),
            pltpu.VMEM((976,), jnp.int32),
            pltpu.VMEM((NPL * 960,), jnp.float32),
            pltpu.SemaphoreType.DMA,
        ],
    )
    def k(d2_hbm, t_hbm, eidx_hbm, planes_hbm,
          d2buf, ttab, cand_k, cand_v, eidx_st, pl_st, osem):
        iota16 = lax.iota(jnp.int32, 16)
        m14 = iota16 < 14
        w = lax.axis_index("s") * 2 + lax.axis_index("c")
        b = w // 4
        pltpu.sync_copy(t_hbm.at[pl.ds(b * (16 * N), 16 * N)], ttab)

        def subchunk(s, _):
            base_row = w * 256 + s * 32
            pltpu.sync_copy(d2_hbm.at[pl.ds(base_row * N, 32 * N)], d2buf)

            def rowbody(r, __):
                rb = r * N
                n_node = (base_row - b * 1024) + r
                m = [None] * 4
                for c in range(64):
                    v = d2buf[pl.ds(rb + c * 16, 16)]
                    q = c // 16
                    m[q] = v if m[q] is None else jnp.minimum(m[q], v)
                sm = [lax.sort(mi) for mi in m]
                ab = _merge16_keys(sm[0], sm[1])
                cd = _merge16_keys(sm[2], sm[3])
                low = _mergelow32_keys(ab, cd)
                t = low[1][13]
                off = jnp.int32(0)
                for c in range(64):
                    v = d2buf[pl.ds(rb + c * 16, 16)]
                    msk = v <= t
                    plsc.store_compressed(cand_k.at[pl.ds(off, 16)], v,
                                          mask=msk)
                    plsc.store_compressed(cand_v.at[pl.ds(off, 16)],
                                          iota16 + (c * 16), mask=msk)
                    cnt = plsc.all_reduce_population_count(msk)
                    off = off + cnt[0]
                k0 = cand_k[pl.ds(0, 16)]
                v0 = cand_v[pl.ds(0, 16)]
                k0, v0 = plsc.sort_key_val(k0, v0)
                k1 = cand_k[pl.ds(16, 16)]
                v1 = cand_v[pl.ds(16, 16)]
                k1 = jnp.where(iota16 < (off - 16), k1, _INF)
                k1, v1 = plsc.sort_key_val(k1, v1)
                state = _merge16_kv(k0, v0, k1, v1)

                def cbody(i, st):
                    s0k, s0v, s1k, s1v = st
                    ck = cand_k[pl.ds(i * 16, 16)]
                    cv = cand_v[pl.ds(i * 16, 16)]
                    ck = jnp.where(iota16 < (off - i * 16), ck, _INF)
                    ck, cv = plsc.sort_key_val(ck, cv)
                    rk, rv = _rev(ck), _rev(cv)
                    cc = s1k <= rk
                    ek = jnp.where(cc, s1k, rk)
                    ev = jnp.where(cc, s1v, rv)
                    ek, ev = plsc.sort_key_val(ek, ev)
                    return _merge16_kv(s0k, s0v, ek, ev)

                nv = (off + 15) // 16
                s0k, s0v, s1k, s1v = lax.fori_loop(2, nv, cbody, state)

                pb = r * 30
                eidx_st[pl.ds(pb, 16)] = s0v
                plsc.store_compressed(eidx_st.at[pl.ds(pb + 16, 16)], s1v,
                                      mask=m14)

                nf = lax.convert_element_type(n_node, jnp.float32)

                def put(p, val0, val1):
                    sb = p * 960 + pb
                    pl_st[pl.ds(sb, 16)] = val0
                    plsc.store_compressed(pl_st.at[pl.ds(sb + 16, 16)],
                                          val1, mask=m14)

                put(0, s0k, s1k)
                put(1, lax.convert_element_type(s0v, jnp.float32) - nf,
                    lax.convert_element_type(s1v, jnp.float32) - nf)
                for c in range(3):
                    base = (9 + c) * N
                    put(2 + c,
                        plsc.load_gather(ttab, [s0v + base]),
                        plsc.load_gather(ttab, [s1v + base]))
                for c in range(3):
                    sv = ttab[pl.ds((9 + c) * N + n_node, 16)][0]
                    vv = lax.broadcast_in_dim(sv, (16,), ())
                    put(5 + c, vv, vv)
                for cc in range(9):
                    base = cc * N
                    put(8 + cc,
                        plsc.load_gather(ttab, [s0v + base]),
                        plsc.load_gather(ttab, [s1v + base]))
                for cc in range(9):
                    sv = ttab[pl.ds(cc * N + n_node, 16)][0]
                    vv = lax.broadcast_in_dim(sv, (16,), ())
                    put(17 + cc, vv, vv)
                return __

            lax.fori_loop(0, 32, rowbody, 0)
            base = w * 7680 + s * 960
            descs = []
            for p in range(NPL):
                descs.append(pltpu.async_copy(
                    pl_st.at[pl.ds(p * 960, 960)],
                    planes_hbm.at[pl.ds(p * NE + base, 960)], osem))
            pltpu.sync_copy(eidx_st.at[pl.ds(0, 960)],
                            eidx_hbm.at[pl.ds(base, 960)])
            for d in descs:
                d.wait()
            return _

        lax.fori_loop(0, 8, subchunk, 0)

    return k(d2_flat, t_flat)


_BES = 512
_BER = 8
_BE = _BER * _BES
_NBLK = NE // _BE


def _feat_body(p_ref, wt_ref, b_ref, g_ref, beta_ref, o_ref, f_ref):
    def pv(c):
        return p_ref[c, 0]

    feats = [None] * 40
    d2v = pv(0)
    didx = pv(1)
    xn = [pv(2 + c) for c in range(3)]
    xs = [pv(5 + c) for c in range(3)]
    On = [pv(8 + c) for c in range(9)]
    Os = [pv(17 + c) for c in range(9)]

    for i in range(8):
        ang = didx * _FREQS[i]
        qq = jnp.round(ang * _INV_2PI)
        ang = (ang - qq * _PI2_HI) - qq * _PI2_LO
        feats[i] = jnp.cos(ang)
        feats[8 + i] = jnp.sin(ang)
    dist = jnp.sqrt(d2v + 1e-6)
    for i in range(16):
        z = (dist - _MUS[i]) / _SIG
        feats[16 + i] = jnp.exp(-(z * z))
    def b16(v):
        return lax.convert_element_type(
            lax.convert_element_type(v, jnp.bfloat16), jnp.float32)

    dx = [xn[c] - xs[c] for c in range(3)]
    dxb = [b16(v) for v in dx]
    Onb = [b16(v) for v in On]
    Osb = [b16(v) for v in Os]
    du = [Osb[3 * v + 0] * dxb[0] + Osb[3 * v + 1] * dxb[1]
          + Osb[3 * v + 2] * dxb[2] for v in range(3)]
    inv = 1.0 / jnp.maximum(
        jnp.sqrt(du[0] * du[0] + du[1] * du[1] + du[2] * du[2]), 1e-12)
    for v in range(3):
        feats[32 + v] = du[v] * inv
    R = [[None] * 3 for _ in range(3)]
    for i in range(3):
        for j in range(3):
            R[i][j] = (Osb[0 + i] * Onb[0 + j] + Osb[3 + i] * Onb[3 + j]
                       + Osb[6 + i] * Onb[6 + j])
    rxx, ryy, rzz = R[0][0], R[1][1], R[2][2]
    mag = [0.5 * jnp.sqrt(jnp.abs(1.0 + (rxx - ryy - rzz))),
           0.5 * jnp.sqrt(jnp.abs(1.0 + (-rxx + ryy - rzz))),
           0.5 * jnp.sqrt(jnp.abs(1.0 + (-rxx - ryy + rzz)))]
    sgn = [jnp.sign(R[2][1] - R[1][2]),
           jnp.sign(R[0][2] - R[2][0]),
           jnp.sign(R[1][0] - R[0][1])]
    q = [sgn[i] * mag[i] for i in range(3)]
    qw = jnp.sqrt(jax.nn.relu(1.0 + (rxx + ryy + rzz))) / 2.0
    q.append(qw)
    inv = 1.0 / jnp.maximum(
        jnp.sqrt(q[0] * q[0] + q[1] * q[1] + q[2] * q[2] + q[3] * q[3]),
        1e-12)
    for v in range(4):
        feats[35 + v] = q[v] * inv

    feats[39] = jnp.zeros_like(d2v)
    for f in range(40):
        f_ref[:, pl.ds(f, 1), :] = feats[f][:, None, :]

    wt = wt_ref[...]
    bb = b_ref[...]
    gg = g_ref[...]
    be = beta_ref[...]
    wtb = lax.convert_element_type(wt, jnp.bfloat16)
    for r in range(_BER):
        fr = f_ref[r]
        frb = lax.convert_element_type(fr, jnp.bfloat16)
        y = lax.dot_general(frb, wtb, (((0,), (0,)), ((), ())),
                            preferred_element_type=jnp.float32)
        y = y + bb
        mu = jnp.mean(y, axis=-1, keepdims=True)
        d = y - mu
        var = jnp.sum(d * d, axis=-1, keepdims=True) / np.float32(127.0)
        sig = jnp.sqrt(var + 1e-6)
        o_ref[0, r] = gg * d / (sig + 1e-6) + be


def _features_linear_ln(planes4, Wt, b, gain, bias):
    return pl.pallas_call(
        _feat_body,
        grid=(_NBLK,),
        in_specs=[
            pl.BlockSpec((NPL, 1, _BER, _BES), lambda i: (0, i, 0, 0)),
            pl.BlockSpec((40, 128), lambda i: (0, 0)),
            pl.BlockSpec((1, 128), lambda i: (0, 0)),
            pl.BlockSpec((1, 128), lambda i: (0, 0)),
            pl.BlockSpec((1, 128), lambda i: (0, 0)),
        ],
        out_specs=pl.BlockSpec((1, _BER, _BES, 128), lambda i: (i, 0, 0, 0)),
        out_shape=jax.ShapeDtypeStruct((_NBLK, _BER, _BES, 128), jnp.float32),
        scratch_shapes=[pltpu.VMEM((_BER, 40, _BES), jnp.float32)],
    )(planes4, Wt, b.reshape(1, 128), gain.reshape(1, 128),
      bias.reshape(1, 128))


def kernel(x, mask, W, b, gain, bias):
    xT = jnp.transpose(x, (0, 2, 1))
    d2, T = _pairwise_d2(x, xT)
    eidx_flat, planes_flat = _topk_sc(d2.reshape(-1), T.reshape(-1))
    edge_idx = eidx_flat.reshape(B, N, TOP_K)
    planes4 = planes_flat.reshape(NPL, _NBLK, _BER, _BES)
    Wt = jnp.concatenate([W.T, jnp.zeros((1, 128), jnp.float32)], axis=0)
    out = _features_linear_ln(planes4, Wt, b, gain, bias)
    return out.reshape(B, N, TOP_K, 128), edge_idx


# speedup vs baseline: 11.5279x; 1.0156x over previous
import functools

import jax
import jax.numpy as jnp
import numpy as np
from jax import lax
from jax.experimental import pallas as pl
from jax.experimental.pallas import tpu as pltpu
from jax.experimental.pallas import tpu_sc as plsc

TOP_K = 30
B, N = 8, 1024
NE = B * N * TOP_K
NPL = 26
_INF = np.float32(np.inf)

_FREQS = (np.arange(0, 16, 2, dtype=np.float32)
          * np.float32(-(np.log(10000.0) / 16.0)))
_FREQS = np.exp(_FREQS).astype(np.float32)
_MUS = np.linspace(0.0, 20.0, 16).astype(np.float32)
_SIG = np.float32(1.25)
_PI2_HI = np.float32(2.0 * np.pi)
_PI2_LO = np.float32(2.0 * np.pi - np.float64(_PI2_HI))
_INV_2PI = np.float32(1.0 / (2.0 * np.pi))


def _safe_inv_norm(n2):
    n = jnp.sqrt(n2)
    return 1.0 / jnp.maximum(n, 1e-12)


def _d2_body(xr_ref, xc_ref, o_ref, t_ref):
    j = pl.program_id(1)
    xr = xr_ref[0]
    xc = xc_ref[0]
    acc = None
    for c in range(3):
        d = xr[:, c:c + 1] - xc[c:c + 1, :]
        acc = d * d if acc is None else acc + d * d
    o_ref[0] = acc

    @pl.when(j == 0)
    def _():
        t_ref[0] = jnp.zeros((16, N), jnp.float32)
        xx = [xc[c:c + 1, :] for c in range(3)]
        dx = [xx[c][:, 1:N] - xx[c][:, 0:N - 1] for c in range(3)]
        inv = _safe_inv_norm(dx[0] * dx[0] + dx[1] * dx[1] + dx[2] * dx[2])
        u = [dx[c] * inv for c in range(3)]
        u2 = [u[c][:, 0:N - 3] for c in range(3)]
        u1 = [u[c][:, 1:N - 2] for c in range(3)]
        cr = [u2[1] * u1[2] - u2[2] * u1[1],
              u2[2] * u1[0] - u2[0] * u1[2],
              u2[0] * u1[1] - u2[1] * u1[0]]
        inv = _safe_inv_norm(cr[0] * cr[0] + cr[1] * cr[1] + cr[2] * cr[2])
        n2v = [cr[c] * inv for c in range(3)]
        o1r = [u2[c] - u1[c] for c in range(3)]
        inv = _safe_inv_norm(o1r[0] * o1r[0] + o1r[1] * o1r[1]
                             + o1r[2] * o1r[2])
        o1 = [o1r[c] * inv for c in range(3)]
        th = [o1[1] * n2v[2] - o1[2] * n2v[1],
              o1[2] * n2v[0] - o1[0] * n2v[2],
              o1[0] * n2v[1] - o1[1] * n2v[0]]
        rows = o1 + n2v + th
        for ri, val in enumerate(rows):
            t_ref[0, ri:ri + 1, 1:N - 2] = val
        for c in range(3):
            t_ref[0, 9 + c:10 + c, :] = xx[c]


def _pairwise_d2(x, xT):
    return pl.pallas_call(
        _d2_body,
        grid=(B, 4),
        in_specs=[
            pl.BlockSpec((1, 256, 3), lambda b, j: (b, j, 0)),
            pl.BlockSpec((1, 3, N), lambda b, j: (b, 0, 0)),
        ],
        out_specs=[
            pl.BlockSpec((1, 256, N), lambda b, j: (b, j, 0)),
            pl.BlockSpec((1, 16, N), lambda b, j: (b, 0, 0)),
        ],
        out_shape=[
            jax.ShapeDtypeStruct((B, N, N), jnp.float32),
            jax.ShapeDtypeStruct((B, 16, N), jnp.float32),
        ],
    )(x, xT)


def _rev(v):
    return lax.rev(v, (0,))


def _merge16_keys(a, b):
    rb = _rev(b)
    lo = jnp.minimum(a, rb)
    hi = jnp.maximum(a, rb)
    return lax.sort(lo), lax.sort(hi)


def _mergelow32_keys(a, b):
    a0, a1 = a
    b0, b1 = b
    c0 = jnp.minimum(a0, _rev(b1))
    c1 = jnp.minimum(a1, _rev(b0))
    d0 = jnp.minimum(c0, c1)
    d1 = jnp.maximum(c0, c1)
    return lax.sort(d0), lax.sort(d1)


def _merge16_kv(ak, av, bk, bv):
    rk, rv = _rev(bk), _rev(bv)
    c = ak <= rk
    lo_k = jnp.where(c, ak, rk)
    lo_v = jnp.where(c, av, rv)
    hi_k = jnp.where(c, rk, ak)
    hi_v = jnp.where(c, rv, av)
    s0 = plsc.sort_key_val(lo_k, lo_v)
    s1 = plsc.sort_key_val(hi_k, hi_v)
    return s0[0], s0[1], s1[0], s1[1]


def _topk_sc(d2_3d, t_flat):
    mesh = plsc.VectorSubcoreMesh(core_axis_name="c", subcore_axis_name="s",
                                  num_cores=2, num_subcores=16)

    @functools.partial(
        pl.kernel,
        out_type=[
            jax.ShapeDtypeStruct((NE,), jnp.int32),
            jax.ShapeDtypeStruct((NPL * NE,), jnp.float32),
        ],
        mesh=mesh,
        compiler_params=pltpu.CompilerParams(needs_layout_passes=False),
        scratch_types=[
            pltpu.VMEM((32 * N,), jnp.float32),
            pltpu.VMEM((16 * N,), jnp.float32),
            pltpu.VMEM((1024,), jnp.float32),
            pltpu.VMEM((1024,), jnp.int32),
            pltpu.VMEM((976,), jnp.int32),
            pltpu.VMEM((NPL * 960,), jnp.float32),
            pltpu.SemaphoreType.DMA,
        ],
    )
    def k(d2_hbm, t_hbm, eidx_hbm, planes_hbm,
          d2buf, ttab, cand_k, cand_v, eidx_st, pl_st, osem):
        iota16 = lax.iota(jnp.int32, 16)
        m14 = iota16 < 14
        w = lax.axis_index("s") * 2 + lax.axis_index("c")
        b = w // 4
        pltpu.sync_copy(t_hbm.at[pl.ds(b * (16 * N), 16 * N)], ttab)

        def subchunk(s, _):
            base_row = w * 256 + s * 32
            descs_in = [
                pltpu.async_copy(
                    d2_hbm.at[b, (base_row - b * 1024) + rr],
                    d2buf.at[pl.ds(rr * N, N)], osem)
                for rr in range(32)
            ]
            for dd in descs_in:
                dd.wait()

            def rowbody(r, __):
                rb = r * N
                n_node = (base_row - b * 1024) + r
                m = [None] * 4
                for c in range(64):
                    v = d2buf[pl.ds(rb + c * 16, 16)]
                    q = c // 16
                    m[q] = v if m[q] is None else jnp.minimum(m[q], v)
                sm = [lax.sort(mi) for mi in m]
                ab = _merge16_keys(sm[0], sm[1])
                cd = _merge16_keys(sm[2], sm[3])
                low = _mergelow32_keys(ab, cd)
                t = low[1][13]
                off = jnp.int32(0)
                for c in range(64):
                    v = d2buf[pl.ds(rb + c * 16, 16)]
                    msk = v <= t
                    plsc.store_compressed(cand_k.at[pl.ds(off, 16)], v,
                                          mask=msk)
                    plsc.store_compressed(cand_v.at[pl.ds(off, 16)],
                                          iota16 + (c * 16), mask=msk)
                    cnt = plsc.all_reduce_population_count(msk)
                    off = off + cnt[0]
                k0 = cand_k[pl.ds(0, 16)]
                v0 = cand_v[pl.ds(0, 16)]
                k0, v0 = plsc.sort_key_val(k0, v0)
                k1 = cand_k[pl.ds(16, 16)]
                v1 = cand_v[pl.ds(16, 16)]
                k1 = jnp.where(iota16 < (off - 16), k1, _INF)
                k1, v1 = plsc.sort_key_val(k1, v1)
                state = _merge16_kv(k0, v0, k1, v1)

                def cbody(i, st):
                    s0k, s0v, s1k, s1v = st
                    ck = cand_k[pl.ds(i * 16, 16)]
                    cv = cand_v[pl.ds(i * 16, 16)]
                    ck = jnp.where(iota16 < (off - i * 16), ck, _INF)
                    ck, cv = plsc.sort_key_val(ck, cv)
                    rk, rv = _rev(ck), _rev(cv)
                    cc = s1k <= rk
                    ek = jnp.where(cc, s1k, rk)
                    ev = jnp.where(cc, s1v, rv)
                    ek, ev = plsc.sort_key_val(ek, ev)
                    return _merge16_kv(s0k, s0v, ek, ev)

                nv = (off + 15) // 16
                s0k, s0v, s1k, s1v = lax.fori_loop(2, nv, cbody, state)

                pb = r * 30
                eidx_st[pl.ds(pb, 16)] = s0v
                plsc.store_compressed(eidx_st.at[pl.ds(pb + 16, 16)], s1v,
                                      mask=m14)

                nf = lax.convert_element_type(n_node, jnp.float32)

                def put(p, val0, val1):
                    sb = p * 960 + pb
                    pl_st[pl.ds(sb, 16)] = val0
                    plsc.store_compressed(pl_st.at[pl.ds(sb + 16, 16)],
                                          val1, mask=m14)

                put(0, s0k, s1k)
                put(1, lax.convert_element_type(s0v, jnp.float32) - nf,
                    lax.convert_element_type(s1v, jnp.float32) - nf)
                for c in range(3):
                    base = (9 + c) * N
                    put(2 + c,
                        plsc.load_gather(ttab, [s0v + base]),
                        plsc.load_gather(ttab, [s1v + base]))
                for c in range(3):
                    sv = ttab[pl.ds((9 + c) * N + n_node, 16)][0]
                    vv = lax.broadcast_in_dim(sv, (16,), ())
                    put(5 + c, vv, vv)
                for cc in range(9):
                    base = cc * N
                    put(8 + cc,
                        plsc.load_gather(ttab, [s0v + base]),
                        plsc.load_gather(ttab, [s1v + base]))
                for cc in range(9):
                    sv = ttab[pl.ds(cc * N + n_node, 16)][0]
                    vv = lax.broadcast_in_dim(sv, (16,), ())
                    put(17 + cc, vv, vv)
                return __

            lax.fori_loop(0, 32, rowbody, 0)
            base = w * 7680 + s * 960
            descs = []
            for p in range(NPL):
                descs.append(pltpu.async_copy(
                    pl_st.at[pl.ds(p * 960, 960)],
                    planes_hbm.at[pl.ds(p * NE + base, 960)], osem))
            pltpu.sync_copy(eidx_st.at[pl.ds(0, 960)],
                            eidx_hbm.at[pl.ds(base, 960)])
            for d in descs:
                d.wait()
            return _

        lax.fori_loop(0, 8, subchunk, 0)

    return k(d2_3d, t_flat)


_BES = 512
_BER = 8
_BE = _BER * _BES
_NBLK = NE // _BE


def _feat_body(p_ref, wt_ref, b_ref, g_ref, beta_ref, o_ref, f_ref):
    def pv(c):
        return p_ref[c, 0]

    feats = [None] * 40
    d2v = pv(0)
    didx = pv(1)
    xn = [pv(2 + c) for c in range(3)]
    xs = [pv(5 + c) for c in range(3)]
    On = [pv(8 + c) for c in range(9)]
    Os = [pv(17 + c) for c in range(9)]

    for i in range(8):
        ang = didx * _FREQS[i]
        qq = jnp.round(ang * _INV_2PI)
        ang = (ang - qq * _PI2_HI) - qq * _PI2_LO
        feats[i] = jnp.cos(ang)
        feats[8 + i] = jnp.sin(ang)
    dist = jnp.sqrt(d2v + 1e-6)
    for i in range(16):
        z = (dist - _MUS[i]) / _SIG
        feats[16 + i] = jnp.exp(-(z * z))
    def b16(v):
        return lax.convert_element_type(
            lax.convert_element_type(v, jnp.bfloat16), jnp.float32)

    dx = [xn[c] - xs[c] for c in range(3)]
    dxb = [b16(v) for v in dx]
    Onb = [b16(v) for v in On]
    Osb = [b16(v) for v in Os]
    du = [Osb[3 * v + 0] * dxb[0] + Osb[3 * v + 1] * dxb[1]
          + Osb[3 * v + 2] * dxb[2] for v in range(3)]
    inv = 1.0 / jnp.maximum(
        jnp.sqrt(du[0] * du[0] + du[1] * du[1] + du[2] * du[2]), 1e-12)
    for v in range(3):
        feats[32 + v] = du[v] * inv
    R = [[None] * 3 for _ in range(3)]
    for i in range(3):
        for j in range(3):
            R[i][j] = (Osb[0 + i] * Onb[0 + j] + Osb[3 + i] * Onb[3 + j]
                       + Osb[6 + i] * Onb[6 + j])
    rxx, ryy, rzz = R[0][0], R[1][1], R[2][2]
    mag = [0.5 * jnp.sqrt(jnp.abs(1.0 + (rxx - ryy - rzz))),
           0.5 * jnp.sqrt(jnp.abs(1.0 + (-rxx + ryy - rzz))),
           0.5 * jnp.sqrt(jnp.abs(1.0 + (-rxx - ryy + rzz)))]
    sgn = [jnp.sign(R[2][1] - R[1][2]),
           jnp.sign(R[0][2] - R[2][0]),
           jnp.sign(R[1][0] - R[0][1])]
    q = [sgn[i] * mag[i] for i in range(3)]
    qw = jnp.sqrt(jax.nn.relu(1.0 + (rxx + ryy + rzz))) / 2.0
    q.append(qw)
    inv = 1.0 / jnp.maximum(
        jnp.sqrt(q[0] * q[0] + q[1] * q[1] + q[2] * q[2] + q[3] * q[3]),
        1e-12)
    for v in range(4):
        feats[35 + v] = q[v] * inv

    feats[39] = jnp.zeros_like(d2v)
    for f in range(40):
        f_ref[:, pl.ds(f, 1), :] = feats[f][:, None, :]

    wt = wt_ref[...]
    bb = b_ref[...]
    gg = g_ref[...]
    be = beta_ref[...]
    wtb = lax.convert_element_type(wt, jnp.bfloat16)
    for r in range(_BER):
        fr = f_ref[r]
        frb = lax.convert_element_type(fr, jnp.bfloat16)
        y = lax.dot_general(frb, wtb, (((0,), (0,)), ((), ())),
                            preferred_element_type=jnp.float32)
        y = y + bb
        mu = jnp.mean(y, axis=-1, keepdims=True)
        d = y - mu
        var = jnp.sum(d * d, axis=-1, keepdims=True) / np.float32(127.0)
        sig = jnp.sqrt(var + 1e-6)
        o_ref[0, r] = gg * d / (sig + 1e-6) + be


def _features_linear_ln(planes4, Wt, b, gain, bias):
    return pl.pallas_call(
        _feat_body,
        grid=(_NBLK,),
        in_specs=[
            pl.BlockSpec((NPL, 1, _BER, _BES), lambda i: (0, i, 0, 0)),
            pl.BlockSpec((40, 128), lambda i: (0, 0)),
            pl.BlockSpec((1, 128), lambda i: (0, 0)),
            pl.BlockSpec((1, 128), lambda i: (0, 0)),
            pl.BlockSpec((1, 128), lambda i: (0, 0)),
        ],
        out_specs=pl.BlockSpec((1, _BER, _BES, 128), lambda i: (i, 0, 0, 0)),
        out_shape=jax.ShapeDtypeStruct((_NBLK, _BER, _BES, 128), jnp.float32),
        scratch_shapes=[pltpu.VMEM((_BER, 40, _BES), jnp.float32)],
    )(planes4, Wt, b.reshape(1, 128), gain.reshape(1, 128),
      bias.reshape(1, 128))


def kernel(x, mask, W, b, gain, bias):
    xT = jnp.transpose(x, (0, 2, 1))
    d2, T = _pairwise_d2(x, xT)
    eidx_flat, planes_flat = _topk_sc(d2, T.reshape(-1))
    edge_idx = eidx_flat.reshape(B, N, TOP_K)
    planes4 = planes_flat.reshape(NPL, _NBLK, _BER, _BES)
    Wt = jnp.concatenate([W.T, jnp.zeros((1, 128), jnp.float32)], axis=0)
    out = _features_linear_ln(planes4, Wt, b, gain, bias)
    return out.reshape(B, N, TOP_K, 128), edge_idx
